# Initial kernel scaffold; baseline (speedup 1.0000x reference)
import jax
import jax.numpy as jnp
from jax.experimental import pallas as pl


def kernel(inputs, indices, output_shape):
    raise NotImplementedError("write your pallas kernel here")


# speedup vs baseline: 9.6336x; 9.6336x over previous
import functools

import jax
import jax.numpy as jnp
from jax import lax
from jax.experimental import pallas as pl
from jax.experimental.pallas import tpu as pltpu
from jax.experimental.pallas import tpu_sc as plsc

B, H, W, C = 4, 192, 192, 96
OH, OW = 384, 384
EPB = H * W * C
PB = OH * OW * C
TOTAL = B * PB

NC, NS = 2, 16
NCH = 8
CH = PB // NCH
PAD = 2048
ACCW = CH + PAD
ET = EPB // NS
BLK = 2048
BR = BLK // 128
NBLK = ET // BLK
OWT = CH // NS
ZW = ACCW // NS
ZB = 13840
ZREP = ZW // ZB


def _sc_scatter_add(val2d, idx2d):
    mesh = plsc.VectorSubcoreMesh(core_axis_name="c", subcore_axis_name="s")

    @functools.partial(
        pl.kernel,
        out_type=jax.ShapeDtypeStruct((TOTAL,), jnp.float32),
        mesh=mesh,
        scratch_types=[
            pltpu.VMEM((BR, 128), jnp.int32),
            pltpu.VMEM((BR, 128), jnp.int32),
            pltpu.VMEM((BR, 128), jnp.float32),
            pltpu.VMEM((ZB,), jnp.float32),
            pltpu.VMEM_SHARED((ACCW,), jnp.float32),
            pltpu.SemaphoreType.DMA,
        ],
    )
    def k(vals_hbm, idx_hbm, out_hbm, idx_raw, idx_loc, val_v, zbuf, acc, sem):
        cid = lax.axis_index("c")
        sid = lax.axis_index("s")

        zero16 = jnp.zeros((16,), jnp.float32)

        def zinit(i, carry):
            zbuf[pl.ds(i * 16, 16)] = zero16
            return carry

        lax.fori_loop(0, ZB // 16, zinit, 0)

        def chunk_body(ck, carry):
            b = cid * 2 + ck // NCH
            r = ck % NCH
            base = r * CH

            def zcopy(i, c2):
                pltpu.sync_copy(zbuf, acc.at[pl.ds(sid * ZW + i * ZB, ZB)])
                return c2

            lax.fori_loop(0, ZREP, zcopy, 0)
            plsc.subcore_barrier()

            row0 = b * (EPB // 128) + sid * (ET // 128)

            def blk_body(g, c3):
                roff = row0 + g * BR
                pltpu.sync_copy(idx_hbm.at[pl.ds(roff, BR)], idx_raw)
                pltpu.sync_copy(vals_hbm.at[pl.ds(roff, BR)], val_v)

                def row_body(j, c4):
                    for kk in range(8):
                        v = idx_raw[j, pl.ds(kk * 16, 16)]
                        local = v - base
                        m = (v >= base) & (local < CH)
                        dump = CH + (v & (PAD - 1))
                        idx_loc[j, pl.ds(kk * 16, 16)] = jnp.where(m, local, dump)
                    pltpu.async_copy(
                        val_v.at[j], acc.at[idx_loc.at[j]], sem, add=True)
                    return c4

                lax.fori_loop(0, BR, row_body, 0)
                pltpu.make_async_copy(vals_hbm.at[pl.ds(0, BR)], val_v, sem).wait()
                return c3

            lax.fori_loop(0, NBLK, blk_body, 0)
            plsc.subcore_barrier()
            pltpu.sync_copy(
                acc.at[pl.ds(sid * OWT, OWT)],
                out_hbm.at[pl.ds(b * PB + base + sid * OWT, OWT)])
            plsc.subcore_barrier()
            return carry

        lax.fori_loop(0, NC * NCH, chunk_body, 0)

    return k(val2d, idx2d)


def kernel(inputs, indices, output_shape):
    del output_shape
    val2d = inputs.reshape(-1, 128)
    idx2d = indices.reshape(-1, 128)
    out = _sc_scatter_add(val2d, idx2d)
    return out.reshape(B, OH, OW, C)


# speedup vs baseline: 18.4783x; 1.9181x over previous
import functools

import jax
import jax.numpy as jnp
from jax import lax
from jax.experimental import pallas as pl
from jax.experimental.pallas import tpu as pltpu
from jax.experimental.pallas import tpu_sc as plsc

B, H, W, C = 4, 192, 192, 96
OH, OW = 384, 384
EPB = H * W * C
PB = OH * OW * C
TOTAL = B * PB

NC, NS = 2, 16
NCH = 8
CH = PB // NCH
PAD = 2048
ACCW = CH + PAD
ET = EPB // NS
BLK = 2048
BR = BLK // 128
NBLK = ET // BLK
NBUF = 3
OWT = CH // NS
ZW = ACCW // NS
ZB = 6920
ZREP = ZW // ZB


def _sc_scatter_add(val2d, idx2d):
    mesh = plsc.VectorSubcoreMesh(core_axis_name="c", subcore_axis_name="s")

    scratch = (
        [pltpu.VMEM((BR, 128), jnp.int32) for _ in range(NBUF)]
        + [pltpu.VMEM((BR, 128), jnp.float32) for _ in range(NBUF)]
        + [
            pltpu.VMEM((ZB,), jnp.float32),
            pltpu.VMEM_SHARED((ACCW,), jnp.float32),
        ]
        + [pltpu.SemaphoreType.DMA for _ in range(2 * NBUF + 1)]
    )

    @functools.partial(
        pl.kernel,
        out_type=jax.ShapeDtypeStruct((TOTAL,), jnp.float32),
        mesh=mesh,
        scratch_types=scratch,
    )
    def k(vals_hbm, idx_hbm, out_hbm, *s):
        idx_raw = s[0:NBUF]
        val_v = s[NBUF:2 * NBUF]
        zbuf = s[2 * NBUF]
        acc = s[2 * NBUF + 1]
        lsem = s[2 * NBUF + 2:2 * NBUF + 2 + NBUF]
        ssem = s[2 * NBUF + 2 + NBUF:2 * NBUF + 2 + 2 * NBUF]
        zsem = s[2 * NBUF + 2 + 2 * NBUF]

        cid = lax.axis_index("c")
        sid = lax.axis_index("s")
        iota16 = lax.iota(jnp.int32, 16)
        zero16 = jnp.zeros((16,), jnp.float32)

        def zinit(i, carry):
            zbuf[pl.ds(i * 16, 16)] = zero16
            return carry

        lax.fori_loop(0, ZB // 16, zinit, 0)

        def chunk_body(ck, carry):
            b = cid * 2 + ck // NCH
            r = ck % NCH
            base = r * CH
            row0 = b * (EPB // 128) + sid * (ET // 128)

            zd = [
                pltpu.async_copy(zbuf, acc.at[pl.ds(sid * ZW + i * ZB, ZB)], zsem)
                for i in range(ZREP)
            ]
            for d in zd:
                d.wait()
            plsc.subcore_barrier()

            def fire_load(g, slot):
                roff = row0 + g * BR
                pltpu.async_copy(idx_hbm.at[pl.ds(roff, BR)], idx_raw[slot],
                                 lsem[slot])
                pltpu.async_copy(vals_hbm.at[pl.ds(roff, BR)], val_v[slot],
                                 lsem[slot])

            fire_load(0, 0)
            fire_load(1, 1)

            def blk3_body(t, c3):
                for slot in range(NBUF):
                    g = t * NBUF + slot
                    pltpu.make_async_copy(
                        idx_hbm.at[pl.ds(0, BR)], idx_raw[slot], lsem[slot]).wait()
                    pltpu.make_async_copy(
                        vals_hbm.at[pl.ds(0, BR)], val_v[slot], lsem[slot]).wait()

                    def row_body(j, c4, slot=slot, g=g):
                        dump = (CH + ((g * BR + j) * 16 & (PAD - 1))) + iota16
                        for kk in range(8):
                            v = idx_raw[slot][j, pl.ds(kk * 16, 16)]
                            local = v - base
                            m = plsc.bitcast(local, jnp.uint32) < jnp.uint32(CH)
                            idx_raw[slot][j, pl.ds(kk * 16, 16)] = (
                                jnp.where(m, local, dump))
                        pltpu.async_copy(val_v[slot].at[j],
                                         acc.at[idx_raw[slot].at[j]],
                                         ssem[slot], add=True)
                        return c4

                    lax.fori_loop(0, BR, row_body, 0)

                    def dbody(j, c5, slot=slot):
                        pltpu.make_async_copy(
                            val_v[slot].at[j], acc.at[idx_raw[slot].at[j]],
                            ssem[slot]).wait()
                        return c5

                    lax.fori_loop(0, BR, dbody, 0)

                    ps = (slot + 2) % NBUF

                    @pl.when(g + 2 < NBLK)
                    def _prefetch(g=g, ps=ps):
                        fire_load(g + 2, ps)
                return c3

            lax.fori_loop(0, NBLK // NBUF, blk3_body, 0)
            plsc.subcore_barrier()
            pltpu.sync_copy(
                acc.at[pl.ds(sid * OWT, OWT)],
                out_hbm.at[pl.ds(b * PB + base + sid * OWT, OWT)])
            plsc.subcore_barrier()
            return carry

        lax.fori_loop(0, NC * NCH, chunk_body, 0)

    return k(val2d, idx2d)


def kernel(inputs, indices, output_shape):
    del output_shape
    val2d = inputs.reshape(-1, 128)
    idx2d = indices.reshape(-1, 128)
    out = _sc_scatter_add(val2d, idx2d)
    return out.reshape(B, OH, OW, C)


# speedup vs baseline: 21.0992x; 1.1418x over previous
import functools

import jax
import jax.numpy as jnp
from jax import lax
from jax.experimental import pallas as pl
from jax.experimental.pallas import tpu as pltpu
from jax.experimental.pallas import tpu_sc as plsc

B, H, W, C = 4, 192, 192, 96
OH, OW = 384, 384
EPB = H * W * C
PB = OH * OW * C
TOTAL = B * PB

NC, NS = 2, 16
NCH = 8
CH = PB // NCH
ACCW = CH
ET = EPB // NS
BLK = 4096
BR = BLK // 128
NBLK = ET // BLK
NBUF = 2
OWT = CH // NS
ZW = ACCW // NS
ZB = 3456
ZREP = ZW // ZB
SENT = -1


def _sc_scatter_add(val2d, idx2d):
    mesh = plsc.VectorSubcoreMesh(core_axis_name="c", subcore_axis_name="s")

    scratch = (
        [pltpu.VMEM((BR, 128), jnp.int32) for _ in range(NBUF)]
        + [pltpu.VMEM((BR, 128), jnp.float32) for _ in range(NBUF)]
        + [
            pltpu.VMEM((ZB,), jnp.float32),
            pltpu.VMEM_SHARED((ACCW,), jnp.float32),
        ]
        + [pltpu.SemaphoreType.DMA for _ in range(2 * NBUF + 1)]
    )

    @functools.partial(
        pl.kernel,
        out_type=jax.ShapeDtypeStruct((TOTAL,), jnp.float32),
        mesh=mesh,
        scratch_types=scratch,
    )
    def k(vals_hbm, idx_hbm, out_hbm, *s):
        idx_raw = s[0:NBUF]
        val_v = s[NBUF:2 * NBUF]
        zbuf = s[2 * NBUF]
        acc = s[2 * NBUF + 1]
        lsem = s[2 * NBUF + 2:2 * NBUF + 2 + NBUF]
        ssem = s[2 * NBUF + 2 + NBUF:2 * NBUF + 2 + 2 * NBUF]
        zsem = s[2 * NBUF + 2 + 2 * NBUF]

        cid = lax.axis_index("c")
        sid = lax.axis_index("s")
        zero16 = jnp.zeros((16,), jnp.float32)

        def zinit(i, carry):
            zbuf[pl.ds(i * 16, 16)] = zero16
            return carry

        lax.fori_loop(0, ZB // 16, zinit, 0)

        def chunk_body(ck, carry):
            b = cid * 2 + ck // NCH
            r = ck % NCH
            base = r * CH
            row0 = b * (EPB // 128) + sid * (ET // 128)

            zd = [
                pltpu.async_copy(zbuf, acc.at[pl.ds(sid * ZW + i * ZB, ZB)], zsem)
                for i in range(ZREP)
            ]
            for d in zd:
                d.wait()
            plsc.subcore_barrier()

            def fire_load(g, slot):
                roff = row0 + g * BR
                pltpu.async_copy(idx_hbm.at[pl.ds(roff, BR)], idx_raw[slot],
                                 lsem[slot])
                pltpu.async_copy(vals_hbm.at[pl.ds(roff, BR)], val_v[slot],
                                 lsem[slot])

            fire_load(0, 0)
            fire_load(1, 1)

            def blk2_body(t, c3):
                for slot in range(NBUF):
                    g = t * NBUF + slot
                    pltpu.make_async_copy(
                        idx_hbm.at[pl.ds(0, BR)], idx_raw[slot], lsem[slot]).wait()
                    pltpu.make_async_copy(
                        vals_hbm.at[pl.ds(0, BR)], val_v[slot], lsem[slot]).wait()

                    def row_body(j, c4, slot=slot):
                        for kk in range(8):
                            v = idx_raw[slot][j, pl.ds(kk * 16, 16)]
                            local = v - base
                            m = plsc.bitcast(local, jnp.uint32) < jnp.uint32(CH)
                            idx_raw[slot][j, pl.ds(kk * 16, 16)] = (
                                jnp.where(m, local, SENT))
                        pltpu.async_copy(
                            val_v[slot].at[j],
                            acc.at[plsc.Indices(idx_raw[slot].at[j],
                                                ignored_value=SENT)],
                            ssem[slot], add=True)
                        return c4

                    lax.fori_loop(0, BR, row_body, 0)

                    def dbody(j, c5, slot=slot):
                        pltpu.make_async_copy(
                            val_v[slot].at[j],
                            acc.at[plsc.Indices(idx_raw[slot].at[j],
                                                ignored_value=SENT)],
                            ssem[slot]).wait()
                        return c5

                    lax.fori_loop(0, BR, dbody, 0)

                    @pl.when(g + 2 < NBLK)
                    def _prefetch(g=g, slot=slot):
                        fire_load(g + 2, slot)
                return c3

            lax.fori_loop(0, NBLK // NBUF, blk2_body, 0)
            plsc.subcore_barrier()
            pltpu.sync_copy(
                acc.at[pl.ds(sid * OWT, OWT)],
                out_hbm.at[pl.ds(b * PB + base + sid * OWT, OWT)])
            plsc.subcore_barrier()
            return carry

        lax.fori_loop(0, NC * NCH, chunk_body, 0)

    return k(val2d, idx2d)


def kernel(inputs, indices, output_shape):
    del output_shape
    val2d = inputs.reshape(-1, 128)
    idx2d = indices.reshape(-1, 128)
    out = _sc_scatter_add(val2d, idx2d)
    return out.reshape(B, OH, OW, C)
